# baseline (speedup 1.0000x reference)
import functools

import jax
import jax.numpy as jnp
from jax import lax
from jax.experimental import pallas as pl
from jax.experimental.pallas import tpu as pltpu
from jax.experimental.pallas import tpu_sc as plsc

MAXLEN = 8192
NROWS = 2 * MAXLEN - 1
D = 1024
SPLIT = MAXLEN - 1

NWORKERS = 32
ROWS_PER_W = 512
CHUNK = 63
NBUF = 2
_STARTS = []
_s = 0
while _s < ROWS_PER_W:
    _STARTS.append(min(_s, ROWS_PER_W - CHUNK))
    _s += CHUNK
NCHUNKS = len(_STARTS)


def _copy_body(table, out, *scratch):
    bufs = scratch[:NBUF]
    isems = scratch[NBUF : 2 * NBUF]
    osems = scratch[2 * NBUF :]
    c = lax.axis_index("c")
    s = lax.axis_index("s")
    wid = s * 2 + c
    is_a = wid < 16
    dst0 = jnp.where(
        is_a,
        jnp.minimum(wid * ROWS_PER_W, SPLIT - ROWS_PER_W),
        SPLIT + (wid - 16) * ROWS_PER_W,
    )
    src0 = dst0 + jnp.where(is_a, MAXLEN, -SPLIT)
    dst0e = dst0 * D
    src0e = src0 * D
    ce = CHUNK * D

    loads = [
        pltpu.make_async_copy(
            table.at[pl.ds(src0e + st * D, ce)], bufs[i % NBUF], isems[i % NBUF]
        )
        for i, st in enumerate(_STARTS)
    ]
    stores = [
        pltpu.make_async_copy(
            bufs[i % NBUF], out.at[pl.ds(dst0e + st * D, ce)], osems[i % NBUF]
        )
        for i, st in enumerate(_STARTS)
    ]

    ahead = NBUF - 1
    waited = set()
    for j in range(min(ahead, NCHUNKS)):
        loads[j].start()
    for i in range(NCHUNKS):
        j = i + ahead
        if j < NCHUNKS:
            if j - NBUF >= 0:
                stores[j - NBUF].wait()
                waited.add(j - NBUF)
            loads[j].start()
        loads[i].wait()
        stores[i].start()
    for i in range(NCHUNKS):
        if i not in waited:
            stores[i].wait()


_shifted_copy = functools.partial(
    pl.kernel,
    mesh=plsc.VectorSubcoreMesh(core_axis_name="c", subcore_axis_name="s"),
    out_type=jax.ShapeDtypeStruct((NROWS * D,), jnp.float32),
    scratch_types=(
        [pltpu.VMEM((CHUNK * D,), jnp.float32)] * NBUF
        + [pltpu.SemaphoreType.DMA] * (2 * NBUF)
    ),
)(_copy_body)


def kernel(x, table):
    del x
    return _shifted_copy(table.reshape(-1)).reshape(NROWS, D)


# speedup vs baseline: 1.0064x; 1.0064x over previous
import functools

import jax
import jax.numpy as jnp
from jax import lax
from jax.experimental import pallas as pl
from jax.experimental.pallas import tpu as pltpu
from jax.experimental.pallas import tpu_sc as plsc

MAXLEN = 8192
NROWS = 2 * MAXLEN - 1
D = 1024
SPLIT = MAXLEN - 1

NWORKERS = 32
ROWS_PER_W = 512
CHUNK = 63
NBUF = 2
_STARTS = []
_s = 0
while _s < ROWS_PER_W:
    _STARTS.append(min(_s, ROWS_PER_W - CHUNK))
    _s += CHUNK
NCHUNKS = len(_STARTS)


def _copy_body(table, out, shared, *scratch):
    isems = scratch[:NBUF]
    osems = scratch[NBUF:]
    c = lax.axis_index("c")
    s = lax.axis_index("s")
    bufs = [shared.at[s, b] for b in range(NBUF)]
    wid = s * 2 + c
    is_a = wid < 16
    dst0 = jnp.where(
        is_a,
        jnp.minimum(wid * ROWS_PER_W, SPLIT - ROWS_PER_W),
        SPLIT + (wid - 16) * ROWS_PER_W,
    )
    src0 = dst0 + jnp.where(is_a, MAXLEN, -SPLIT)
    dst0e = dst0 * D
    src0e = src0 * D
    ce = CHUNK * D

    loads = [
        pltpu.make_async_copy(
            table.at[pl.ds(src0e + st * D, ce)], bufs[i % NBUF], isems[i % NBUF]
        )
        for i, st in enumerate(_STARTS)
    ]
    stores = [
        pltpu.make_async_copy(
            bufs[i % NBUF], out.at[pl.ds(dst0e + st * D, ce)], osems[i % NBUF]
        )
        for i, st in enumerate(_STARTS)
    ]

    ahead = NBUF - 1
    waited = set()
    for j in range(min(ahead, NCHUNKS)):
        loads[j].start()
    for i in range(NCHUNKS):
        j = i + ahead
        if j < NCHUNKS:
            if j - NBUF >= 0:
                stores[j - NBUF].wait()
                waited.add(j - NBUF)
            loads[j].start()
        loads[i].wait()
        stores[i].start()
    for i in range(NCHUNKS):
        if i not in waited:
            stores[i].wait()


_shifted_copy = functools.partial(
    pl.kernel,
    mesh=plsc.VectorSubcoreMesh(core_axis_name="c", subcore_axis_name="s"),
    out_type=jax.ShapeDtypeStruct((NROWS * D,), jnp.float32),
    scratch_types=(
        [pltpu.VMEM_SHARED((16, NBUF, CHUNK * D), jnp.float32)]
        + [pltpu.SemaphoreType.DMA] * (2 * NBUF)
    ),
)(_copy_body)


def kernel(x, table):
    del x
    return _shifted_copy(table.reshape(-1)).reshape(NROWS, D)


# speedup vs baseline: 2.5144x; 2.4983x over previous
import functools

import jax
import jax.numpy as jnp
from jax import lax
from jax.experimental import pallas as pl
from jax.experimental.pallas import tpu as pltpu
from jax.experimental.pallas import tpu_sc as plsc

MAXLEN = 8192
NROWS = 2 * MAXLEN - 1
D = 1024
SEAM = MAXLEN - 8

ROWS_PER_W = 256
CHUNK = 56
NBUF = 2
_STARTS = []
_s = 0
while _s < ROWS_PER_W:
    _STARTS.append(min(_s, ROWS_PER_W - CHUNK))
    _s += CHUNK
NCHUNKS = len(_STARTS)


def _sc_body(table, out, *scratch):
    bufs = scratch[:NBUF]
    isems = scratch[NBUF : 2 * NBUF]
    osems = scratch[2 * NBUF :]
    c = lax.axis_index("c")
    s = lax.axis_index("s")
    w = s * 2 + c
    dst0 = pl.multiple_of(jnp.minimum(w * ROWS_PER_W, SEAM - ROWS_PER_W), 8)
    src0 = pl.multiple_of(dst0 + MAXLEN, 8)

    loads = [
        pltpu.make_async_copy(
            table.at[pl.ds(src0 + st, CHUNK)], bufs[i % NBUF], isems[i % NBUF]
        )
        for i, st in enumerate(_STARTS)
    ]
    stores = [
        pltpu.make_async_copy(
            bufs[i % NBUF], out.at[pl.ds(dst0 + st, CHUNK)], osems[i % NBUF]
        )
        for i, st in enumerate(_STARTS)
    ]

    ahead = NBUF - 1
    waited = set()
    for j in range(min(ahead, NCHUNKS)):
        loads[j].start()
    for i in range(NCHUNKS):
        j = i + ahead
        if j < NCHUNKS:
            if j - NBUF >= 0:
                stores[j - NBUF].wait()
                waited.add(j - NBUF)
            loads[j].start()
        loads[i].wait()
        stores[i].start()
    for i in range(NCHUNKS):
        if i not in waited:
            stores[i].wait()


_sc_half = functools.partial(
    pl.kernel,
    mesh=plsc.VectorSubcoreMesh(core_axis_name="c", subcore_axis_name="s"),
    out_type=jax.ShapeDtypeStruct((NROWS, D), jnp.float32),
    scratch_types=(
        [pltpu.VMEM((CHUNK, D), jnp.float32)] * NBUF
        + [pltpu.SemaphoreType.DMA] * (2 * NBUF)
    ),
)(_sc_body)


def _seam_body(prev, a, b, o):
    del prev
    o[...] = jnp.concatenate([a[0:7], b[0:1]], axis=0)


_tc_seam = pl.pallas_call(
    _seam_body,
    grid=(1,),
    in_specs=[
        pl.BlockSpec(memory_space=pl.ANY),
        pl.BlockSpec((8, D), lambda i: (2 * MAXLEN // 8 - 1, 0)),
        pl.BlockSpec((8, D), lambda i: (0, 0)),
    ],
    out_specs=pl.BlockSpec((8, D), lambda i: (SEAM // 8, 0)),
    out_shape=jax.ShapeDtypeStruct((NROWS, D), jnp.float32),
    input_output_aliases={0: 0},
)


def _main_body(prev, a, b, o):
    del prev
    o[...] = jnp.concatenate([a[1:256], b[0:1]], axis=0)


_tc_half = pl.pallas_call(
    _main_body,
    grid=(32,),
    in_specs=[
        pl.BlockSpec(memory_space=pl.ANY),
        pl.BlockSpec((256, D), lambda k: (k, 0)),
        pl.BlockSpec((8, D), lambda k: (32 * (k + 1), 0)),
    ],
    out_specs=pl.BlockSpec((256, D), lambda k: (32 + k, 0)),
    out_shape=jax.ShapeDtypeStruct((NROWS, D), jnp.float32),
    input_output_aliases={0: 0},
)


def kernel(x, table):
    del x
    p = _sc_half(table)
    p = _tc_seam(p, table, table)
    return _tc_half(p, table, table)


# speedup vs baseline: 2.8219x; 1.1223x over previous
import functools

import jax
import jax.numpy as jnp
from jax import lax
from jax.experimental import pallas as pl
from jax.experimental.pallas import tpu as pltpu
from jax.experimental.pallas import tpu_sc as plsc

MAXLEN = 8192
NROWS = 2 * MAXLEN - 1
D = 1024
SC_ROWS = 7680
TCB = 512

ROWS_PER_W = SC_ROWS // 32
CHUNK = 56
NBUF = 2
_STARTS = []
_s = 0
while _s < ROWS_PER_W:
    _STARTS.append(min(_s, ROWS_PER_W - CHUNK))
    _s += CHUNK
NCHUNKS = len(_STARTS)


def _sc_body(table, out, *scratch):
    bufs = scratch[:NBUF]
    isems = scratch[NBUF : 2 * NBUF]
    osems = scratch[2 * NBUF :]
    c = lax.axis_index("c")
    s = lax.axis_index("s")
    w = s * 2 + c
    dst0 = pl.multiple_of(w * ROWS_PER_W, 8)
    src0 = pl.multiple_of(dst0 + MAXLEN, 8)

    loads = [
        pltpu.make_async_copy(
            table.at[pl.ds(src0 + st, CHUNK)], bufs[i % NBUF], isems[i % NBUF]
        )
        for i, st in enumerate(_STARTS)
    ]
    stores = [
        pltpu.make_async_copy(
            bufs[i % NBUF], out.at[pl.ds(dst0 + st, CHUNK)], osems[i % NBUF]
        )
        for i, st in enumerate(_STARTS)
    ]

    ahead = NBUF - 1
    waited = set()
    for j in range(min(ahead, NCHUNKS)):
        loads[j].start()
    for i in range(NCHUNKS):
        j = i + ahead
        if j < NCHUNKS:
            if j - NBUF >= 0:
                stores[j - NBUF].wait()
                waited.add(j - NBUF)
            loads[j].start()
        loads[i].wait()
        stores[i].start()
    for i in range(NCHUNKS):
        if i not in waited:
            stores[i].wait()


_sc_half = functools.partial(
    pl.kernel,
    mesh=plsc.VectorSubcoreMesh(core_axis_name="c", subcore_axis_name="s"),
    out_type=jax.ShapeDtypeStruct((NROWS, D), jnp.float32),
    scratch_types=(
        [pltpu.VMEM((CHUNK, D), jnp.float32)] * NBUF
        + [pltpu.SemaphoreType.DMA] * (2 * NBUF)
    ),
)(_sc_body)


def _tc_body(prev, a, b, o):
    del prev
    k = pl.program_id(0)

    @pl.when(k == 0)
    def _seam():
        o[...] = jnp.concatenate([a[0 : TCB - 1], b[0:1]], axis=0)

    @pl.when(k > 0)
    def _shift():
        o[...] = jnp.concatenate([a[1:TCB], b[0:1]], axis=0)


_tc_half = pl.pallas_call(
    _tc_body,
    grid=(17,),
    in_specs=[
        pl.BlockSpec(memory_space=pl.ANY),
        pl.BlockSpec((TCB, D), lambda k: (jnp.where(k == 0, 31, k - 1), 0)),
        pl.BlockSpec((8, D), lambda k: (64 * k, 0)),
    ],
    out_specs=pl.BlockSpec((TCB, D), lambda k: (15 + k, 0)),
    out_shape=jax.ShapeDtypeStruct((NROWS, D), jnp.float32),
    input_output_aliases={0: 0},
)


def kernel(x, table):
    del x
    return _tc_half(_sc_half(table), table, table)


# speedup vs baseline: 3.0951x; 1.0968x over previous
import functools

import jax
import jax.numpy as jnp
from jax import lax
from jax.experimental import pallas as pl
from jax.experimental.pallas import tpu as pltpu
from jax.experimental.pallas import tpu_sc as plsc

MAXLEN = 8192
NROWS = 2 * MAXLEN - 1
D = 1024
SC_ROWS = 7168
TCB = 1024

ROWS_PER_W = SC_ROWS // 32
CHUNK = 56
NBUF = 2
_STARTS = []
_s = 0
while _s < ROWS_PER_W:
    _STARTS.append(min(_s, ROWS_PER_W - CHUNK))
    _s += CHUNK
NCHUNKS = len(_STARTS)


def _sc_body(table, out, *scratch):
    bufs = scratch[:NBUF]
    isems = scratch[NBUF : 2 * NBUF]
    osems = scratch[2 * NBUF :]
    c = lax.axis_index("c")
    s = lax.axis_index("s")
    w = s * 2 + c
    dst0 = pl.multiple_of(w * ROWS_PER_W, 8)
    src0 = pl.multiple_of(dst0 + MAXLEN, 8)

    loads = [
        pltpu.make_async_copy(
            table.at[pl.ds(src0 + st, CHUNK)], bufs[i % NBUF], isems[i % NBUF]
        )
        for i, st in enumerate(_STARTS)
    ]
    stores = [
        pltpu.make_async_copy(
            bufs[i % NBUF], out.at[pl.ds(dst0 + st, CHUNK)], osems[i % NBUF]
        )
        for i, st in enumerate(_STARTS)
    ]

    ahead = NBUF - 1
    waited = set()
    for j in range(min(ahead, NCHUNKS)):
        loads[j].start()
    for i in range(NCHUNKS):
        j = i + ahead
        if j < NCHUNKS:
            if j - NBUF >= 0:
                stores[j - NBUF].wait()
                waited.add(j - NBUF)
            loads[j].start()
        loads[i].wait()
        stores[i].start()
    for i in range(NCHUNKS):
        if i not in waited:
            stores[i].wait()


_sc_half = functools.partial(
    pl.kernel,
    mesh=plsc.VectorSubcoreMesh(core_axis_name="c", subcore_axis_name="s"),
    out_type=jax.ShapeDtypeStruct((NROWS, D), jnp.float32),
    scratch_types=(
        [pltpu.VMEM((CHUNK, D), jnp.float32)] * NBUF
        + [pltpu.SemaphoreType.DMA] * (2 * NBUF)
    ),
)(_sc_body)


def _tc_body(prev, a, b, o):
    del prev
    k = pl.program_id(0)

    @pl.when(k == 0)
    def _seam():
        o[...] = jnp.concatenate([a[0 : TCB - 1], b[0:1]], axis=0)

    @pl.when(k > 0)
    def _shift():
        o[...] = jnp.concatenate([a[1:TCB], b[0:1]], axis=0)


_tc_half = pl.pallas_call(
    _tc_body,
    grid=(9,),
    in_specs=[
        pl.BlockSpec(memory_space=pl.ANY),
        pl.BlockSpec((TCB, D), lambda k: (jnp.where(k == 0, 15, k - 1), 0)),
        pl.BlockSpec((8, D), lambda k: (128 * k, 0)),
    ],
    out_specs=pl.BlockSpec((TCB, D), lambda k: (7 + k, 0)),
    out_shape=jax.ShapeDtypeStruct((NROWS, D), jnp.float32),
    input_output_aliases={0: 0},
)


def kernel(x, table):
    del x
    return _tc_half(_sc_half(table), table, table)


# speedup vs baseline: 3.1106x; 1.0050x over previous
import functools

import jax
import jax.numpy as jnp
from jax import lax
from jax.experimental import pallas as pl
from jax.experimental.pallas import tpu as pltpu
from jax.experimental.pallas import tpu_sc as plsc

MAXLEN = 8192
NROWS = 2 * MAXLEN - 1
D = 1024
SC_ROWS = 7168
TCB = 1024

ROWS_PER_W = SC_ROWS // 32
CHUNK = 32
NBUF = 3
_STARTS = []
_s = 0
while _s < ROWS_PER_W:
    _STARTS.append(min(_s, ROWS_PER_W - CHUNK))
    _s += CHUNK
NCHUNKS = len(_STARTS)


def _sc_body(table, out, *scratch):
    bufs = scratch[:NBUF]
    isems = scratch[NBUF : 2 * NBUF]
    osems = scratch[2 * NBUF :]
    c = lax.axis_index("c")
    s = lax.axis_index("s")
    w = s * 2 + c
    dst0 = pl.multiple_of(w * ROWS_PER_W, 8)
    src0 = pl.multiple_of(dst0 + MAXLEN, 8)

    loads = [
        pltpu.make_async_copy(
            table.at[pl.ds(src0 + st, CHUNK)], bufs[i % NBUF], isems[i % NBUF]
        )
        for i, st in enumerate(_STARTS)
    ]
    stores = [
        pltpu.make_async_copy(
            bufs[i % NBUF], out.at[pl.ds(dst0 + st, CHUNK)], osems[i % NBUF]
        )
        for i, st in enumerate(_STARTS)
    ]

    ahead = NBUF - 1
    waited = set()
    for j in range(min(ahead, NCHUNKS)):
        loads[j].start()
    for i in range(NCHUNKS):
        j = i + ahead
        if j < NCHUNKS:
            if j - NBUF >= 0:
                stores[j - NBUF].wait()
                waited.add(j - NBUF)
            loads[j].start()
        loads[i].wait()
        stores[i].start()
    for i in range(NCHUNKS):
        if i not in waited:
            stores[i].wait()


_sc_half = functools.partial(
    pl.kernel,
    mesh=plsc.VectorSubcoreMesh(core_axis_name="c", subcore_axis_name="s"),
    out_type=jax.ShapeDtypeStruct((NROWS, D), jnp.float32),
    scratch_types=(
        [pltpu.VMEM((CHUNK, D), jnp.float32)] * NBUF
        + [pltpu.SemaphoreType.DMA] * (2 * NBUF)
    ),
)(_sc_body)


def _tc_body(prev, a, b, o):
    del prev
    k = pl.program_id(0)

    @pl.when(k == 0)
    def _seam():
        o[...] = jnp.concatenate([a[0 : TCB - 1], b[0:1]], axis=0)

    @pl.when(k > 0)
    def _shift():
        o[...] = jnp.concatenate([a[1:TCB], b[0:1]], axis=0)


_tc_half = pl.pallas_call(
    _tc_body,
    grid=(9,),
    in_specs=[
        pl.BlockSpec(memory_space=pl.ANY),
        pl.BlockSpec((TCB, D), lambda k: (jnp.where(k == 0, 15, k - 1), 0)),
        pl.BlockSpec((8, D), lambda k: (128 * k, 0)),
    ],
    out_specs=pl.BlockSpec((TCB, D), lambda k: (7 + k, 0)),
    out_shape=jax.ShapeDtypeStruct((NROWS, D), jnp.float32),
    input_output_aliases={0: 0},
)


def kernel(x, table):
    del x
    return _tc_half(_sc_half(table), table, table)
